# baseline (speedup 1.0000x reference)
import functools

import jax
import jax.numpy as jnp
from jax import lax
from jax.experimental import pallas as pl
from jax.experimental.pallas import tpu as pltpu
from jax.experimental.pallas import tpu_sc as plsc

T = 16384
H = 2048
E = 16

NUM_WORKERS = 32
CHUNKS = 1


TILE_T = 1024


def _gate_body(h_ref, w_ref, o_ref, ot_ref, acc_ref):
    k = pl.program_id(1)
    part = jnp.dot(h_ref[...], w_ref[...],
                   preferred_element_type=jnp.float32)

    @pl.when(k == 0)
    def _():
        acc_ref[...] = part

    @pl.when(k == 1)
    def _():
        acc = jnp.clip(acc_ref[...] + part, -50.0, 50.0)
        o_ref[...] = acc
        ot_ref[...] = acc.T


def _gate_logits(hidden_states, gate_wt, chunk_t, c):
    nblk = chunk_t // TILE_T
    return pl.pallas_call(
        _gate_body,
        grid=(nblk, 2),
        in_specs=[
            pl.BlockSpec((TILE_T, H // 2), lambda i, k: (c * nblk + i, k)),
            pl.BlockSpec((H // 2, E), lambda i, k: (k, 0)),
        ],
        out_specs=[
            pl.BlockSpec((TILE_T, E), lambda i, k: (i, 0)),
            pl.BlockSpec((E, TILE_T), lambda i, k: (0, i)),
        ],
        out_shape=[
            jax.ShapeDtypeStruct((chunk_t, E), jnp.float32),
            jax.ShapeDtypeStruct((E, chunk_t), jnp.float32),
        ],
        scratch_shapes=[pltpu.VMEM((TILE_T, E), jnp.float32)],
    )(hidden_states, gate_wt)


def _make_route_body(per_w):
    groups = per_w // 16

    def _route_body(lgt_hbm, wt_hbm, it_hbm, lgt_v, wt_v, it_v):
        wid = lax.axis_index("s") * 2 + lax.axis_index("c")
        base = wid * per_w
        pltpu.sync_copy(lgt_hbm.at[:, pl.ds(base, per_w)], lgt_v)

        def group(g, carry):
            off = g * 16
            m1 = lgt_v[0, pl.ds(off, 16)]
            i1 = jnp.zeros((16,), jnp.int32)
            m2 = jnp.full((16,), -jnp.inf, jnp.float32)
            i2 = jnp.zeros((16,), jnp.int32)
            for e in range(1, E):
                e_vec = jnp.full((16,), e, jnp.int32)
                v = lgt_v[e, pl.ds(off, 16)]
                gt1 = v > m1
                gt2 = v > m2
                m2 = jnp.where(gt1, m1, jnp.where(gt2, v, m2))
                i2 = jnp.where(gt1, i1, jnp.where(gt2, e_vec, i2))
                m1 = jnp.where(gt1, v, m1)
                i1 = jnp.where(gt1, e_vec, i1)
            e2 = jnp.exp(m2 - m1)
            denom = 1.0 + e2
            wt_v[0, pl.ds(off, 16)] = 1.0 / denom
            wt_v[1, pl.ds(off, 16)] = e2 / denom
            it_v[0, pl.ds(off, 16)] = i1
            it_v[1, pl.ds(off, 16)] = i2
            return carry

        lax.fori_loop(0, groups, group, 0)

        pltpu.sync_copy(wt_v, wt_hbm.at[:, pl.ds(base, per_w)])
        pltpu.sync_copy(it_v, it_hbm.at[:, pl.ds(base, per_w)])

    return _route_body


def _route(logits_t, chunk_t):
    per_w = chunk_t // NUM_WORKERS
    mesh = plsc.VectorSubcoreMesh(core_axis_name="c", subcore_axis_name="s")
    f = functools.partial(
        pl.kernel,
        mesh=mesh,
        out_type=(
            jax.ShapeDtypeStruct((2, chunk_t), jnp.float32),
            jax.ShapeDtypeStruct((2, chunk_t), jnp.int32),
        ),
        scratch_types=[
            pltpu.VMEM((E, per_w), jnp.float32),
            pltpu.VMEM((2, per_w), jnp.float32),
            pltpu.VMEM((2, per_w), jnp.int32),
        ],
    )(_make_route_body(per_w))
    return f(logits_t)


@jax.jit
def _run(hidden_states, gate_wt):
    chunk_t = T // CHUNKS
    lg, wt, it = [], [], []
    for c in range(CHUNKS):
        logits_c, logits_t_c = _gate_logits(hidden_states, gate_wt,
                                            chunk_t, c)
        wt_c, it_c = _route(logits_t_c, chunk_t)
        lg.append(logits_c)
        wt.append(wt_c)
        it.append(it_c)
    logits = lg[0] if CHUNKS == 1 else jnp.concatenate(lg, axis=0)
    weights = (wt[0].T if CHUNKS == 1
               else jnp.concatenate([w.T for w in wt], axis=0))
    indices = (it[0].T if CHUNKS == 1
               else jnp.concatenate([i.T for i in it], axis=0))
    return logits, weights, indices


def kernel(hidden_states, gate_w):
    return _run(hidden_states, gate_w.T)


# speedup vs baseline: 1.2010x; 1.2010x over previous
import functools

import jax
import jax.numpy as jnp
from jax import lax
from jax.experimental import pallas as pl
from jax.experimental.pallas import tpu as pltpu
from jax.experimental.pallas import tpu_sc as plsc

T = 16384
H = 2048
E = 16

NUM_WORKERS = 32
CHUNKS = 1


TILE_T = 1024


def _gate_body(h_ref, w_ref, o_ref, ot_ref):
    acc = jnp.dot(h_ref[...], w_ref[...],
                  preferred_element_type=jnp.float32)
    acc = jnp.clip(acc, -50.0, 50.0)
    o_ref[...] = acc
    ot_ref[...] = acc.T


def _gate_logits(hidden_states, gate_wt, chunk_t, c):
    nblk = chunk_t // TILE_T
    return pl.pallas_call(
        _gate_body,
        grid=(nblk,),
        in_specs=[
            pl.BlockSpec((TILE_T, H), lambda i: (c * nblk + i, 0)),
            pl.BlockSpec((H, E), lambda i: (0, 0)),
        ],
        out_specs=[
            pl.BlockSpec((TILE_T, E), lambda i: (i, 0)),
            pl.BlockSpec((E, TILE_T), lambda i: (0, i)),
        ],
        out_shape=[
            jax.ShapeDtypeStruct((chunk_t, E), jnp.float32),
            jax.ShapeDtypeStruct((E, chunk_t), jnp.float32),
        ],
    )(hidden_states, gate_wt)


def _make_route_body(per_w):
    groups = per_w // 16

    def _route_body(lgt_hbm, wt_hbm, it_hbm, lgt_v, wt_v, it_v):
        wid = lax.axis_index("s") * 2 + lax.axis_index("c")
        base = wid * per_w
        pltpu.sync_copy(lgt_hbm.at[:, pl.ds(base, per_w)], lgt_v)

        def group(g, carry):
            off = g * 16
            m1 = lgt_v[0, pl.ds(off, 16)]
            i1 = jnp.zeros((16,), jnp.int32)
            m2 = jnp.full((16,), -jnp.inf, jnp.float32)
            i2 = jnp.zeros((16,), jnp.int32)
            for e in range(1, E):
                e_vec = jnp.full((16,), e, jnp.int32)
                v = lgt_v[e, pl.ds(off, 16)]
                gt1 = v > m1
                gt2 = v > m2
                m2 = jnp.where(gt1, m1, jnp.where(gt2, v, m2))
                i2 = jnp.where(gt1, i1, jnp.where(gt2, e_vec, i2))
                m1 = jnp.where(gt1, v, m1)
                i1 = jnp.where(gt1, e_vec, i1)
            e2 = jnp.exp(m2 - m1)
            denom = 1.0 + e2
            wt_v[0, pl.ds(off, 16)] = 1.0 / denom
            wt_v[1, pl.ds(off, 16)] = e2 / denom
            it_v[0, pl.ds(off, 16)] = i1
            it_v[1, pl.ds(off, 16)] = i2
            return carry

        lax.fori_loop(0, groups, group, 0)

        pltpu.sync_copy(wt_v, wt_hbm.at[:, pl.ds(base, per_w)])
        pltpu.sync_copy(it_v, it_hbm.at[:, pl.ds(base, per_w)])

    return _route_body


def _route(logits_t, chunk_t):
    per_w = chunk_t // NUM_WORKERS
    mesh = plsc.VectorSubcoreMesh(core_axis_name="c", subcore_axis_name="s")
    f = functools.partial(
        pl.kernel,
        mesh=mesh,
        out_type=(
            jax.ShapeDtypeStruct((2, chunk_t), jnp.float32),
            jax.ShapeDtypeStruct((2, chunk_t), jnp.int32),
        ),
        scratch_types=[
            pltpu.VMEM((E, per_w), jnp.float32),
            pltpu.VMEM((2, per_w), jnp.float32),
            pltpu.VMEM((2, per_w), jnp.int32),
        ],
    )(_make_route_body(per_w))
    return f(logits_t)


@jax.jit
def _run(hidden_states, gate_wt):
    chunk_t = T // CHUNKS
    lg, wt, it = [], [], []
    for c in range(CHUNKS):
        logits_c, logits_t_c = _gate_logits(hidden_states, gate_wt,
                                            chunk_t, c)
        wt_c, it_c = _route(logits_t_c, chunk_t)
        lg.append(logits_c)
        wt.append(wt_c)
        it.append(it_c)
    logits = lg[0] if CHUNKS == 1 else jnp.concatenate(lg, axis=0)
    weights = (wt[0].T if CHUNKS == 1
               else jnp.concatenate([w.T for w in wt], axis=0))
    indices = (it[0].T if CHUNKS == 1
               else jnp.concatenate([i.T for i in it], axis=0))
    return logits, weights, indices


def kernel(hidden_states, gate_w):
    return _run(hidden_states, gate_w.T)
